# baseline (device time: 9192 ns/iter reference)
import jax
import jax.numpy as jnp
from jax import lax
from jax.experimental import pallas as pl
from jax.experimental.pallas import tpu as pltpu

N_COLS_GLOBAL = 2048
N_CHUNKS = 4


def kernel(x):
    m, n = x.shape
    rows, cols = m // 128, 128
    m_c = m // N_CHUNKS
    rows_c = rows // N_CHUNKS

    def body(x_ref, out_ref, acc_ref, recv_ref, send_sems, recv_sems):
        i = pl.program_id(0)
        my_x = lax.axis_index("x")
        my_y = lax.axis_index("y")
        nbr = (my_x, 1 - my_y)
        barrier = pltpu.get_barrier_semaphore()

        @pl.when(i == 0)
        def _():
            pl.semaphore_signal(
                barrier, inc=1, device_id=nbr, device_id_type=pl.DeviceIdType.MESH
            )

        x3 = x_ref[...].reshape(rows_c, cols, n)
        acc_ref[i] = jnp.sum(x3, axis=2)

        @pl.when(i == 0)
        def _():
            pl.semaphore_wait(barrier, 1)

        rdma = pltpu.make_async_remote_copy(
            src_ref=acc_ref.at[i],
            dst_ref=recv_ref.at[i],
            send_sem=send_sems.at[i],
            recv_sem=recv_sems.at[i],
            device_id=nbr,
            device_id_type=pl.DeviceIdType.MESH,
        )
        rdma.start()

        @pl.when(i == N_CHUNKS - 1)
        def _():
            for c in range(N_CHUNKS):
                drain = pltpu.make_async_remote_copy(
                    src_ref=acc_ref.at[c],
                    dst_ref=recv_ref.at[c],
                    send_sem=send_sems.at[c],
                    recv_sem=recv_sems.at[c],
                    device_id=nbr,
                    device_id_type=pl.DeviceIdType.MESH,
                )
                drain.wait()

            acc = acc_ref[...].reshape(rows, cols)
            recv = recv_ref[...].reshape(rows, cols)
            total = (acc + recv) * (1.0 / N_COLS_GLOBAL)
            blocks = jnp.broadcast_to(total[:, None, :], (rows, cols, cols))
            expanded = blocks.reshape(m, cols)
            ridx = lax.broadcasted_iota(jnp.int32, (m, cols), 0)
            cidx = lax.broadcasted_iota(jnp.int32, (m, cols), 1)
            picked = jnp.where(cidx == ridx % cols, expanded, 0.0)
            out_ref[...] = jnp.sum(picked, axis=1, keepdims=True)

    return pl.pallas_call(
        body,
        grid=(N_CHUNKS,),
        out_shape=jax.ShapeDtypeStruct((m, 1), jnp.float32),
        in_specs=[
            pl.BlockSpec((m_c, n), lambda i: (i, 0), memory_space=pltpu.VMEM)
        ],
        out_specs=pl.BlockSpec((m, 1), lambda i: (0, 0), memory_space=pltpu.VMEM),
        scratch_shapes=[
            pltpu.VMEM((N_CHUNKS, rows_c, cols), jnp.float32),
            pltpu.VMEM((N_CHUNKS, rows_c, cols), jnp.float32),
            pltpu.SemaphoreType.DMA((N_CHUNKS,)),
            pltpu.SemaphoreType.DMA((N_CHUNKS,)),
        ],
        compiler_params=pltpu.CompilerParams(collective_id=0),
    )(x)
